# baseline (speedup 1.0000x reference)
import functools

import jax
import jax.numpy as jnp
from jax import lax
from jax.experimental import pallas as pl
from jax.experimental.pallas import tpu as pltpu
from jax.experimental.pallas import tpu_sc as plsc

N = 10000
E = 160000
D = 256

NC = 2
NS = 16
NW = NC * NS
L = 16

SLAB = 1600
NSLAB = E // SLAB
G = 64
NBLK = -(-N // 8)
BPT = -(-NBLK // NW)
AROWS = BPT * 8
TRASH = AROWS


def _mm_body(x_ref, wt_ref, b_ref, o_ref):
    o_ref[...] = (
        jnp.dot(x_ref[...], wt_ref[...], preferred_element_type=jnp.float32)
        + b_ref[...]
    )


def _linear(layer_input, wt, b):
    blk = 400
    return pl.pallas_call(
        _mm_body,
        grid=(N // blk,),
        in_specs=[
            pl.BlockSpec((blk, D), lambda i: (i, 0)),
            pl.BlockSpec((D, D), lambda i: (0, 0)),
            pl.BlockSpec((1, D), lambda i: (0, 0)),
        ],
        out_specs=pl.BlockSpec((blk, D), lambda i: (i, 0)),
        out_shape=jax.ShapeDtypeStruct((N, D), jnp.float32),
    )(layer_input, wt, b.reshape(1, D))


_mesh = plsc.VectorSubcoreMesh(
    core_axis_name="c", subcore_axis_name="s", num_cores=NC, num_subcores=NS
)


@functools.partial(
    pl.kernel,
    out_type=jax.ShapeDtypeStruct((N, D), jnp.float32),
    mesh=_mesh,
    compiler_params=pltpu.CompilerParams(needs_layout_passes=False),
    scratch_types=[
        pltpu.VMEM((SLAB,), jnp.int32),
        pltpu.VMEM((SLAB,), jnp.int32),
        pltpu.VMEM((SLAB,), jnp.float32),
        pltpu.VMEM((SLAB + G,), jnp.int32),
        pltpu.VMEM((SLAB + G,), jnp.int32),
        pltpu.VMEM((SLAB + G,), jnp.float32),
        pltpu.VMEM((G,), jnp.int32),
        pltpu.VMEM((G, D), jnp.float32),
        pltpu.VMEM((AROWS + 1, D), jnp.float32),
        pltpu.SemaphoreType.DMA,
    ],
)
def _sc_spmm(x_hbm, src_hbm, dst_hbm, w_hbm, out_hbm,
             sslab, dslab, wslab, csrc, cdst, cw, idxg, rows_v, acc, sem):
    c = lax.axis_index("c")
    s = lax.axis_index("s")
    wid = s * NC + c

    def _zrow(i, _):
        for j in range(D // L):
            acc[i, pl.ds(j * L, L)] = jnp.zeros((L,), jnp.float32)
        return 0
    lax.fori_loop(0, AROWS + 1, _zrow, 0)

    iota = lax.iota(jnp.int32, L)

    def _slab(si, _):
        e0 = pl.multiple_of(si * SLAB, 8)
        pltpu.sync_copy(src_hbm.at[pl.ds(e0, SLAB)], sslab)
        pltpu.sync_copy(dst_hbm.at[pl.ds(e0, SLAB)], dslab)
        pltpu.sync_copy(w_hbm.at[pl.ds(e0, SLAB)], wslab)

        def _scan(g, off):
            d = dslab[pl.ds(g * L, L)]
            m = ((d >> 3) & (NW - 1)) == wid
            lr = ((d >> 8) << 3) | (d & 7)
            plsc.store_compressed(cdst.at[pl.ds(off, L)], lr, mask=m)
            plsc.store_compressed(csrc.at[pl.ds(off, L)],
                                  sslab[pl.ds(g * L, L)], mask=m)
            plsc.store_compressed(cw.at[pl.ds(off, L)],
                                  wslab[pl.ds(g * L, L)], mask=m)
            return off + plsc.all_reduce_population_count(m)[0]
        cnt = lax.fori_loop(0, SLAB // L, _scan, 0)

        pend = ((cnt + (G - 1)) // G) * G
        for t in range(G // L):
            idx = iota + (cnt + t * L)
            mpad = idx < pend
            plsc.store_compressed(csrc.at[pl.ds(cnt + t * L, L)],
                                  jnp.zeros((L,), jnp.int32), mask=mpad)
            plsc.store_compressed(cdst.at[pl.ds(cnt + t * L, L)],
                                  jnp.full((L,), TRASH, jnp.int32), mask=mpad)
            plsc.store_compressed(cw.at[pl.ds(cnt + t * L, L)],
                                  jnp.zeros((L,), jnp.float32), mask=mpad)

        def _gchunk(gi, _):
            goff = gi * G
            for t in range(G // L):
                idxg[pl.ds(t * L, L)] = csrc[pl.ds(goff + t * L, L)]
            pltpu.async_copy(x_hbm.at[idxg], rows_v, sem).wait()

            def _e16(h, _):
                k0 = goff + h * L
                dvec = cdst[pl.ds(k0, L)]
                wvec = cw[pl.ds(k0, L)]
                for i in range(L):
                    r = dvec[i]
                    wb = jnp.full((L,), wvec[i])
                    kr = h * L + i
                    for j in range(D // L):
                        acc[r, pl.ds(j * L, L)] = (
                            acc[r, pl.ds(j * L, L)]
                            + rows_v[kr, pl.ds(j * L, L)] * wb)
                return 0
            lax.fori_loop(0, G // L, _e16, 0)
            return 0
        lax.fori_loop(0, pend // G, _gchunk, 0)
        return 0
    lax.fori_loop(0, NSLAB, _slab, 0)

    for k in range(BPT):
        blk = wid + NW * k
        @pl.when(blk < NBLK)
        def _():
            pltpu.sync_copy(
                acc.at[pl.ds(k * 8, 8)],
                out_hbm.at[pl.ds(pl.multiple_of(blk * 8, 8), 8)])


def kernel(layer_input, edge_index, edge_weight, W, b):
    x = _linear(layer_input, W.T, b)
    return _sc_spmm(x, edge_index[1], edge_index[0], edge_weight)
